# Initial kernel scaffold; baseline (speedup 1.0000x reference)
import jax
import jax.numpy as jnp
from jax.experimental import pallas as pl


def kernel(query, key, value, mask):
    raise NotImplementedError("write your pallas kernel here")


# speedup vs baseline: 1.9898x; 1.9898x over previous
import functools

import jax
import jax.numpy as jnp
from jax.experimental import pallas as pl
from jax.experimental.pallas import tpu as pltpu

MBLK = 64


def _attn_kernel(q_ref, k_ref, v_ref, m_ref, o_ref, allow_ref, *, tk):
    qi = pl.program_id(1)
    S = k_ref.shape[2]
    D = k_ref.shape[3]
    nb = S // MBLK

    q = q_ref[0, 0]
    mask_slab = m_ref[0].astype(jnp.float32)

    colsum = jnp.sum(mask_slab, axis=0, keepdims=True)
    blk_ids = jax.lax.broadcasted_iota(jnp.int32, (nb, S), 0)
    col_ids = jax.lax.broadcasted_iota(jnp.int32, (nb, S), 1)
    E = (col_ids // MBLK == blk_ids).astype(jnp.float32)
    blocksum = jax.lax.dot_general(
        colsum, E, (((1,), (1,)), ((), ())),
        preferred_element_type=jnp.float32)
    active = (blocksum > 0).astype(jnp.float32)
    allow_ref[...] = jax.lax.dot_general(
        active, E, (((1,), (0,)), ((), ())),
        preferred_element_type=jnp.float32)

    row_ids = qi * MBLK + jax.lax.broadcasted_iota(jnp.int32, (MBLK, tk), 0)
    col_iota = jax.lax.broadcasted_iota(jnp.int32, (MBLK, tk), 1)

    def body(j, carry):
        m_i, l_i, acc = carry
        k = k_ref[0, 0, pl.ds(j * tk, tk), :]
        v = v_ref[0, 0, pl.ds(j * tk, tk), :]
        s = jax.lax.dot_general(
            q, k, (((1,), (1,)), ((), ())),
            preferred_element_type=jnp.float32)
        cols = j * tk + col_iota
        ballow = allow_ref[:, pl.ds(j * tk, tk)] > 0.5
        allow = ballow & (cols <= row_ids)
        s = jnp.where(allow, s, -1e37)
        m_chunk = jnp.max(s, axis=1, keepdims=True)
        m_new = jnp.maximum(m_i, m_chunk)
        p = jnp.exp(s - m_new) * allow.astype(jnp.float32)
        alpha = jnp.exp(m_i - m_new)
        l_new = l_i * alpha + jnp.sum(p, axis=1, keepdims=True)
        acc_new = acc * alpha + jax.lax.dot_general(
            p, v, (((1,), (0,)), ((), ())),
            preferred_element_type=jnp.float32)
        return m_new, l_new, acc_new

    n_chunks = qi * MBLK // tk + 1
    m0 = jnp.full((MBLK, 1), -1e37, jnp.float32)
    l0 = jnp.zeros((MBLK, 1), jnp.float32)
    acc0 = jnp.zeros((MBLK, D), jnp.float32)
    m_f, l_f, acc_f = jax.lax.fori_loop(0, n_chunks, body, (m0, l0, acc0))

    out = jnp.where(l_f > 0, acc_f / jnp.maximum(l_f, 1e-30), 0.0)
    o_ref[0, 0] = out


@jax.jit
def kernel(query, key, value, mask):
    B, H, S, D = query.shape
    nb = S // MBLK
    tk = min(512, S)
    grid = (H, nb)
    out = pl.pallas_call(
        functools.partial(_attn_kernel, tk=tk),
        grid=grid,
        in_specs=[
            pl.BlockSpec((1, 1, MBLK, D), lambda h, i: (0, h, i, 0)),
            pl.BlockSpec((1, 1, S, D), lambda h, i: (0, h, 0, 0)),
            pl.BlockSpec((1, 1, S, D), lambda h, i: (0, h, 0, 0)),
            pl.BlockSpec((1, MBLK, S), lambda h, i: (h, i, 0)),
        ],
        out_specs=pl.BlockSpec((1, 1, MBLK, D), lambda h, i: (0, h, i, 0)),
        out_shape=jax.ShapeDtypeStruct((B, H, S, D), jnp.float32),
        scratch_shapes=[pltpu.VMEM((1, S), jnp.float32)],
    )(query, key, value, mask)
    return out


# speedup vs baseline: 5.1016x; 2.5639x over previous
import functools

import jax
import jax.numpy as jnp
from jax.experimental import pallas as pl
from jax.experimental.pallas import tpu as pltpu

MBLK = 64


def _attn_kernel(q_ref, k_ref, v_ref, m_ref, o_ref, allow_ref, *, qt, tk):
    qi = pl.program_id(1)
    S = k_ref.shape[2]
    D = k_ref.shape[3]
    nb = S // MBLK
    ng = qt // MBLK

    q = q_ref[0, 0]
    mask_slab = m_ref[0].astype(jnp.float32)

    g_rows = jax.lax.broadcasted_iota(jnp.int32, (ng, qt), 0)
    g_cols = jax.lax.broadcasted_iota(jnp.int32, (ng, qt), 1)
    G = (g_cols // MBLK == g_rows).astype(jnp.float32)
    colsum = jax.lax.dot_general(
        G, mask_slab, (((1,), (0,)), ((), ())),
        preferred_element_type=jnp.float32)
    blk_ids = jax.lax.broadcasted_iota(jnp.int32, (nb, S), 0)
    col_ids = jax.lax.broadcasted_iota(jnp.int32, (nb, S), 1)
    E = (col_ids // MBLK == blk_ids).astype(jnp.float32)
    blocksum = jax.lax.dot_general(
        colsum, E, (((1,), (1,)), ((), ())),
        preferred_element_type=jnp.float32)
    active = (blocksum > 0).astype(jnp.float32)
    allow_ref[...] = jax.lax.dot_general(
        active, E, (((1,), (0,)), ((), ())),
        preferred_element_type=jnp.float32)

    row_ids = qi * qt + jax.lax.broadcasted_iota(jnp.int32, (qt, tk), 0)
    col_iota = jax.lax.broadcasted_iota(jnp.int32, (qt, tk), 1)

    def body(j, carry):
        m_i, l_i, acc = carry
        k = k_ref[0, 0, pl.ds(j * tk, tk), :]
        v = v_ref[0, 0, pl.ds(j * tk, tk), :]
        s = jax.lax.dot_general(
            q, k, (((1,), (1,)), ((), ())),
            preferred_element_type=jnp.float32)
        allow_g = allow_ref[:, pl.ds(j * tk, tk)]
        ballow = jnp.concatenate(
            [jnp.broadcast_to(allow_g[g:g + 1, :], (MBLK, tk))
             for g in range(ng)], axis=0) > 0.5
        cols = j * tk + col_iota
        allow = ballow & (cols <= row_ids)
        s = jnp.where(allow, s, -1e37)
        m_chunk = jnp.max(s, axis=1, keepdims=True)
        m_new = jnp.maximum(m_i, m_chunk)
        p = jnp.exp(s - m_new) * allow.astype(jnp.float32)
        alpha = jnp.exp(m_i - m_new)
        l_new = l_i * alpha + jnp.sum(p, axis=1, keepdims=True)
        acc_new = acc * alpha + jax.lax.dot_general(
            p, v, (((1,), (0,)), ((), ())),
            preferred_element_type=jnp.float32)
        return m_new, l_new, acc_new

    n_chunks = qi * qt // tk + 1
    m0 = jnp.full((qt, 1), -1e37, jnp.float32)
    l0 = jnp.zeros((qt, 1), jnp.float32)
    acc0 = jnp.zeros((qt, D), jnp.float32)
    m_f, l_f, acc_f = jax.lax.fori_loop(0, n_chunks, body, (m0, l0, acc0))

    out = jnp.where(l_f > 0, acc_f / jnp.maximum(l_f, 1e-30), 0.0)
    o_ref[0, 0] = out


@jax.jit
def kernel(query, key, value, mask):
    B, H, S, D = query.shape
    qt = min(256, S)
    tk = min(512, S)
    ng = qt // MBLK
    grid = (H, S // qt)
    out = pl.pallas_call(
        functools.partial(_attn_kernel, qt=qt, tk=tk),
        grid=grid,
        in_specs=[
            pl.BlockSpec((1, 1, qt, D), lambda h, i: (0, h, i, 0)),
            pl.BlockSpec((1, 1, S, D), lambda h, i: (0, h, 0, 0)),
            pl.BlockSpec((1, 1, S, D), lambda h, i: (0, h, 0, 0)),
            pl.BlockSpec((1, qt, S), lambda h, i: (h, i, 0)),
        ],
        out_specs=pl.BlockSpec((1, 1, qt, D), lambda h, i: (0, h, i, 0)),
        out_shape=jax.ShapeDtypeStruct((B, H, S, D), jnp.float32),
        scratch_shapes=[pltpu.VMEM((ng, S), jnp.float32)],
    )(query, key, value, mask)
    return out
